# baseline (speedup 1.0000x reference)
import functools

import jax
import jax.numpy as jnp
import numpy as np
from jax import lax
from jax.experimental import pallas as pl
from jax.experimental.pallas import tpu as pltpu
from jax.experimental.pallas import tpu_sc as plsc

NSC = 2
NSUB = 16
LANES = 16
CHUNK = 512
SUB = 128

_PACK_CH = np.arange(256) % 16 + (np.arange(256) // 128) * 16
_PACK_GRP = (np.arange(256) % 128) // 16


def _pack_vec(v):
    return v[_PACK_CH]


def _pack_mat(m):
    return m[_PACK_CH[:, None], _PACK_CH[None, :]] * (
        _PACK_GRP[:, None] == _PACK_GRP[None, :]).astype(jnp.float32)


def _pad8(a):
    return jnp.concatenate(
        [a[None, :], jnp.zeros((7, a.shape[0]), jnp.float32)], axis=0)


def _prolh_body(x_ref, w_ref, b_ref, o_ref):
    y = jnp.dot(x_ref[...], w_ref[...], preferred_element_type=jnp.float32)
    y = y + b_ref[0:1, :]
    o_ref[...] = y * jax.nn.sigmoid(y)


def _prologue_nodes(x, w, b):
    n = x.shape[0]
    blk = 2000
    return pl.pallas_call(
        _prolh_body,
        grid=(n // blk,),
        in_specs=[
            pl.BlockSpec((blk, 128), lambda i: (i, 0)),
            pl.BlockSpec((128, 32), lambda i: (0, 0)),
            pl.BlockSpec((8, 32), lambda i: (0, 0)),
        ],
        out_specs=pl.BlockSpec((blk, 32), lambda i: (i, 0)),
        out_shape=jax.ShapeDtypeStruct((n, 32), jnp.float32),
    )(x, w, _pad8(b))


def _prole_body(ea_ref, s_ref, par_ref, wbig_ref, wo_ref, po_ref):
    wcat = jnp.dot(ea_ref[...], s_ref[...], preferred_element_type=jnp.float32)
    wcat = wcat + par_ref[0:1, :]
    wcat = wcat * jax.nn.sigmoid(wcat)
    wo_ref[0] = wcat[:, 0:128]
    wo_ref[1] = wcat[:, 128:256]
    pn = jnp.dot(wcat, wbig_ref[...], preferred_element_type=jnp.float32)
    pn = pn + par_ref[1:2, :]
    po_ref[0] = pn[:, 0:128]
    po_ref[1] = pn[:, 128:256]


def _prologue_edges(ea8, ew_lin, eb_lin, ew0, eb0):
    rows = ea8.shape[0]
    blk = 1000
    s = (ew_lin[0, _PACK_CH][None, :]
         * (jnp.arange(8)[:, None] == _PACK_GRP[None, :])).astype(jnp.float32)
    par = jnp.concatenate([
        _pack_vec(eb_lin)[None, :], _pack_vec(eb0)[None, :],
        jnp.zeros((6, 256), jnp.float32)], axis=0)
    return pl.pallas_call(
        _prole_body,
        grid=(rows // blk,),
        in_specs=[
            pl.BlockSpec((blk, 8), lambda i: (i, 0)),
            pl.BlockSpec((8, 256), lambda i: (0, 0)),
            pl.BlockSpec((8, 256), lambda i: (0, 0)),
            pl.BlockSpec((256, 256), lambda i: (0, 0)),
        ],
        out_specs=[pl.BlockSpec((2, blk, 128), lambda i: (0, i, 0))] * 2,
        out_shape=[jax.ShapeDtypeStruct((2, rows, 128), jnp.float32)] * 2,
    )(ea8, s, par, _pack_mat(ew0))


def _node1_body(h_ref, w_ref, b_ref, x1_ref, t_ref, r_ref):
    y = jnp.dot(h_ref[...], w_ref[...], preferred_element_type=jnp.float32)
    y = y + b_ref[0:1, :]
    x1_ref[...] = y[:, 0:32]
    t_ref[0] = y[:, 32:64]
    t_ref[1] = y[:, 64:96]
    r_ref[0] = y[:, 96:112]
    r_ref[1] = y[:, 112:128]


def _node1(h, wnode, bnode):
    n = h.shape[0]
    blk = 2000
    return pl.pallas_call(
        _node1_body,
        grid=(n // blk,),
        in_specs=[
            pl.BlockSpec((blk, 32), lambda i: (i, 0)),
            pl.BlockSpec((32, 128), lambda i: (0, 0)),
            pl.BlockSpec((8, 128), lambda i: (0, 0)),
        ],
        out_specs=[
            pl.BlockSpec((blk, 32), lambda i: (i, 0)),
            pl.BlockSpec((2, blk, 32), lambda i: (0, i, 0)),
            pl.BlockSpec((2, blk, 16), lambda i: (0, i, 0)),
        ],
        out_shape=[
            jax.ShapeDtypeStruct((n, 32), jnp.float32),
            jax.ShapeDtypeStruct((2, n, 32), jnp.float32),
            jax.ShapeDtypeStruct((2, n, 16), jnp.float32),
        ],
    )(h, wnode, bnode)


def _node2a_body(x1_ref, a_ref, invb_ref, u_ref, acc_ref):
    agg = jnp.concatenate([a_ref[0], a_ref[1]], axis=1)
    u = x1_ref[...] + agg * invb_ref[...]
    u_ref[...] = u
    acc_ref[0, 0:1, :] = jnp.sum(u, axis=0, keepdims=True)
    acc_ref[0, 1:2, :] = jnp.sum(u * u, axis=0, keepdims=True)


def _node2b_body(u_ref, h_ref, par_ref, out_ref):
    z = u_ref[...] * par_ref[0:1, :] + par_ref[1:2, :]
    out_ref[...] = h_ref[...] + z * jax.nn.sigmoid(z)


def _node2(x1, agg2, invb, h, g, b):
    n = x1.shape[0]
    blk = 2000
    nb = n // blk
    u, acc = pl.pallas_call(
        _node2a_body,
        grid=(nb,),
        in_specs=[
            pl.BlockSpec((blk, 32), lambda i: (i, 0)),
            pl.BlockSpec((2, blk, 16), lambda i: (0, i, 0)),
            pl.BlockSpec((blk, 32), lambda i: (i, 0)),
        ],
        out_specs=[
            pl.BlockSpec((blk, 32), lambda i: (i, 0)),
            pl.BlockSpec((1, 8, 32), lambda i: (i, 0, 0)),
        ],
        out_shape=[
            jax.ShapeDtypeStruct((n, 32), jnp.float32),
            jax.ShapeDtypeStruct((nb, 8, 32), jnp.float32),
        ],
    )(x1, agg2, invb)
    tot = jnp.sum(acc[:, 0:2, :], axis=0)
    mu = tot[0] / n
    var = tot[1] / n - mu * mu
    inv_sig = lax.rsqrt(var + 1e-5)
    scale = inv_sig * g
    shift = b - mu * scale
    par = jnp.concatenate([scale[None, :], shift[None, :],
                           jnp.zeros((6, 32), jnp.float32)], axis=0)
    return pl.pallas_call(
        _node2b_body,
        grid=(nb,),
        in_specs=[
            pl.BlockSpec((blk, 32), lambda i: (i, 0)),
            pl.BlockSpec((blk, 32), lambda i: (i, 0)),
            pl.BlockSpec((8, 32), lambda i: (0, 0)),
        ],
        out_specs=pl.BlockSpec((blk, 32), lambda i: (i, 0)),
        out_shape=jax.ShapeDtypeStruct((n, 32), jnp.float32),
    )(u, h, par)


def _passb_body(w_ref, p_ref, par_ref, wbig_ref, wo_ref, po_ref):
    wcat = jnp.concatenate([w_ref[0], w_ref[1]], axis=1)
    zcat = jnp.concatenate([p_ref[0], p_ref[1]], axis=1)
    zcat = zcat * par_ref[0:1, :] + par_ref[1:2, :]
    wn = wcat + zcat * jax.nn.sigmoid(zcat)
    wo_ref[0] = wn[:, 0:128]
    wo_ref[1] = wn[:, 128:256]
    pn = jnp.dot(wn, wbig_ref[...], preferred_element_type=jnp.float32)
    pn = pn + par_ref[2:3, :]
    po_ref[0] = pn[:, 0:128]
    po_ref[1] = pn[:, 128:256]


def _passb(w_st, pre_st, scale, shift, ew, eb):
    rows = w_st.shape[1]
    blk = 1000
    par = jnp.concatenate([
        _pack_vec(scale)[None, :], _pack_vec(shift)[None, :],
        _pack_vec(eb)[None, :], jnp.zeros((5, 256), jnp.float32)], axis=0)
    return pl.pallas_call(
        _passb_body,
        grid=(rows // blk,),
        in_specs=[
            pl.BlockSpec((2, blk, 128), lambda i: (0, i, 0)),
            pl.BlockSpec((2, blk, 128), lambda i: (0, i, 0)),
            pl.BlockSpec((8, 256), lambda i: (0, 0)),
            pl.BlockSpec((256, 256), lambda i: (0, 0)),
        ],
        out_specs=[pl.BlockSpec((2, blk, 128), lambda i: (0, i, 0))] * 2,
        out_shape=[jax.ShapeDtypeStruct((2, rows, 128), jnp.float32)] * 2,
    )(w_st, pre_st, par, _pack_mat(ew))


def _sc_mesh():
    return plsc.VectorSubcoreMesh(core_axis_name="c", subcore_axis_name="s",
                                  num_cores=NSC, num_subcores=NSUB)


_SC_PARAMS = pltpu.CompilerParams(use_tc_tiling_on_sc=False)


def _npad(n):
    return ((n // NSUB + 7) // 8 * 8) * NSUB


def _make_sc_passA(n, e):
    total_chunks = e // CHUNK
    n_loop = (total_chunks + NSUB - 1) // NSUB
    nidx = CHUNK // SUB
    rows = CHUNK // 8
    npad = _npad(n)
    nhalf = npad // 2
    dump = nhalf

    scratch = [
        pltpu.VMEM((nidx, SUB), jnp.int32),
        pltpu.VMEM((nidx, SUB), jnp.int32),
        pltpu.VMEM((rows, 128), jnp.float32),
        pltpu.VMEM((rows, 128), jnp.float32),
        pltpu.VMEM((CHUNK, 2 * LANES), jnp.float32),
        pltpu.VMEM((CHUNK, LANES), jnp.float32),
        pltpu.VMEM((CHUNK, LANES), jnp.float32),
        pltpu.VMEM((rows, 128), jnp.float32),
        pltpu.VMEM((2, LANES), jnp.float32),
        pltpu.VMEM_SHARED((nhalf + 8, LANES), jnp.float32),
        pltpu.SemaphoreType.DMA,
        pltpu.SemaphoreType.DMA,
    ]
    out_type = [
        jax.ShapeDtypeStruct((NSC, 2, nhalf, LANES), jnp.float32),
        jax.ShapeDtypeStruct((NSC, NSUB, 2, LANES), jnp.float32),
        jax.ShapeDtypeStruct((NSC, e // 8, 128), jnp.float32),
        jax.ShapeDtypeStruct((NSC, e, LANES), jnp.float32),
    ]

    @functools.partial(pl.kernel, out_type=out_type, mesh=_sc_mesh(),
                       scratch_types=scratch, compiler_params=_SC_PARAMS)
    def sc_passA(src2d_hbm, dst2d_hbm, w_hbm, p_hbm, t_hbm, r_hbm, zeros_hbm,
                 agg_out, stats_out, pre_out, c_out,
                 src_v, dst_v, w_v, p_v, t_v, r_v, c_v, pre_v, st_v,
                 agg_s, sem, sem2):
        c = lax.axis_index("c")
        s = lax.axis_index("s")

        slab = nhalf // NSUB
        soff = pl.multiple_of(s * slab, slab)

        st_v[0, :] = jnp.zeros((LANES,), jnp.float32)
        st_v[1, :] = jnp.zeros((LANES,), jnp.float32)

        @pl.loop(0, 2)
        def _(h2):
            pltpu.sync_copy(zeros_hbm.at[pl.ds(soff, slab)],
                            agg_s.at[pl.ds(soff, slab)])

            @pl.when(s == 0)
            def _():
                pltpu.sync_copy(zeros_hbm.at[pl.ds(0, 8)],
                                agg_s.at[pl.ds(nhalf, 8)])
            plsc.subcore_barrier()
            lo = h2 * nhalf

            @pl.loop(0, n_loop)
            def _(k):
                m = s + k * NSUB

                @pl.when(m < total_chunks)
                def _():
                    row0 = pl.multiple_of(m * nidx, nidx)
                    prow = pl.multiple_of(m * rows, rows)
                    base = pl.multiple_of(m * CHUNK, CHUNK)

                    pltpu.sync_copy(src2d_hbm.at[pl.ds(row0, nidx)], src_v)

                    @pl.when(h2 == 0)
                    def _():
                        pltpu.sync_copy(dst2d_hbm.at[pl.ds(row0, nidx)],
                                        dst_v)
                        pltpu.sync_copy(w_hbm.at[c, pl.ds(prow, rows)], w_v)
                        pltpu.sync_copy(p_hbm.at[c, pl.ds(prow, rows)], p_v)
                        cps = []
                        for j in range(nidx):
                            cps.append(pltpu.async_copy(
                                t_hbm.at[c].at[dst_v.at[j]],
                                t_v.at[pl.ds(j * SUB, SUB)], sem))
                            cps.append(pltpu.async_copy(
                                r_hbm.at[c].at[src_v.at[j]],
                                r_v.at[pl.ds(j * SUB, SUB)], sem2))
                        for cp in cps:
                            cp.wait()

                        def row_body(r, car2):
                            es, eq = car2
                            for jj in range(8):
                                i = r * 8 + jj
                                w0 = w_v[r, pl.ds(jj * LANES, LANES)]
                                sg = 1.0 / (1.0 + jnp.exp(-w0))
                                g2 = t_v[i, pl.ds(0, LANES)]
                                c_v[i, :] = sg * g2
                                g4 = t_v[i, pl.ds(LANES, LANES)]
                                pre = (p_v[r, pl.ds(jj * LANES, LANES)]
                                       + r_v[i, :] + g4)
                                pre_v[r, pl.ds(jj * LANES, LANES)] = pre
                                es = es + pre
                                eq = eq + pre * pre
                            return es, eq

                        zero = jnp.zeros((LANES,), jnp.float32)
                        es, eq = lax.fori_loop(0, rows, row_body,
                                               (zero, zero))
                        st_v[0, :] += es
                        st_v[1, :] += eq
                        pltpu.sync_copy(
                            pre_v, pre_out.at[c, pl.ds(prow, rows)])
                        pltpu.sync_copy(
                            c_v, c_out.at[c, pl.ds(base, CHUNK)])

                    @pl.when(h2 == 1)
                    def _():
                        pltpu.sync_copy(
                            c_out.at[c, pl.ds(base, CHUNK)], c_v)

                    for j in range(nidx):
                        for l0 in range(0, SUB, LANES):
                            iv = src_v[j, pl.ds(l0, LANES)]
                            ok = jnp.logical_and(iv >= lo, iv < lo + nhalf)
                            dst_v[j, pl.ds(l0, LANES)] = jnp.where(
                                ok, iv - lo, dump)
                    for j in range(nidx):
                        pltpu.sync_copy(c_v.at[pl.ds(j * SUB, SUB)],
                                        agg_s.at[dst_v.at[j]], add=True)

            plsc.subcore_barrier()
            pltpu.sync_copy(agg_s.at[pl.ds(soff, slab)],
                            agg_out.at[c, h2, pl.ds(soff, slab)])
            plsc.subcore_barrier()

        pltpu.sync_copy(st_v, stats_out.at[c, s])

    return sc_passA


def _make_sc_count(n, e):
    NOCT = 8
    npad = _npad(n)
    nq = ((npad // NOCT + 127) // 128) * 128
    total_chunks = e // CHUNK
    n_loop = (total_chunks + NSC * NSUB - 1) // (NSC * NSUB)
    nidx = CHUNK // SUB

    scratch = [
        pltpu.VMEM((nidx, SUB), jnp.int32),
        pltpu.VMEM((nidx, SUB), jnp.int32),
        pltpu.VMEM((SUB, LANES), jnp.float32),
        pltpu.VMEM_SHARED((nq + 8, LANES), jnp.float32),
    ]
    out_type = [jax.ShapeDtypeStruct((NSC, NOCT, nq, LANES), jnp.float32)]

    @functools.partial(pl.kernel, out_type=out_type, mesh=_sc_mesh(),
                       scratch_types=scratch, compiler_params=_SC_PARAMS)
    def sc_count(src2d_hbm, ones_hbm, zeros_hbm, cnt_out,
                 src_v, loc_v, ones_v, agg_s):
        c = lax.axis_index("c")
        s = lax.axis_index("s")
        wid = c * NSUB + s
        pltpu.sync_copy(ones_hbm, ones_v)

        @pl.loop(0, NOCT)
        def _(q):
            @pl.when(s == 0)
            def _():
                pltpu.sync_copy(zeros_hbm.at[pl.ds(0, nq + 8)],
                                agg_s.at[pl.ds(0, nq + 8)])
            plsc.subcore_barrier()
            qlo = q * nq

            @pl.loop(0, n_loop)
            def _(k):
                m = wid + k * (NSC * NSUB)

                @pl.when(m < total_chunks)
                def _():
                    moff = pl.multiple_of(m * nidx, nidx)
                    pltpu.sync_copy(src2d_hbm.at[pl.ds(moff, nidx)], src_v)
                    for j in range(nidx):
                        for l0 in range(0, SUB, LANES):
                            iv = src_v[j, pl.ds(l0, LANES)]
                            ok = jnp.logical_and(iv >= qlo, iv < qlo + nq)
                            loc = jnp.where(ok, iv - qlo, nq)
                            loc_v[j, pl.ds(l0, LANES)] = loc
                    for j in range(nidx):
                        pltpu.sync_copy(ones_v,
                                        agg_s.at[loc_v.at[j]], add=True)

            plsc.subcore_barrier()

            @pl.when(s == 0)
            def _():
                pltpu.sync_copy(agg_s.at[pl.ds(0, nq)], cnt_out.at[c, q])
            plsc.subcore_barrier()

    return sc_count, nq


def kernel(x, edge_index, edge_attr, v_lin0_w, v_lin0_b, v1_w, v1_b, v2_w, v2_b, v3_w, v3_b, v4_w, v4_b, e_lin0_w, e_lin0_b, e0_w, e0_b, v_bn_g, v_bn_b, e_bn_g, e_bn_b):
    n = x.shape[0]
    e = edge_index.shape[1]
    d = v1_w.shape[0]

    src2d = edge_index[0].reshape(e // SUB, SUB)
    dst2d = edge_index[1].reshape(e // SUB, SUB)
    npad = _npad(n)
    zeros_n = jnp.zeros((npad, LANES), jnp.float32)
    ones_sub = jnp.ones((SUB, LANES), jnp.float32)

    h = _prologue_nodes(x, v_lin0_w, v_lin0_b)
    ea8 = edge_attr.reshape(e // 8, 8)
    w_st, p_st = _prologue_edges(ea8, e_lin0_w, e_lin0_b, e0_w[0], e0_b[0])

    sc_count, nq = _make_sc_count(n, e)
    cnt_o = sc_count(src2d, ones_sub, zeros_n)[0]
    cnt = (cnt_o[0, :, :, 0] + cnt_o[1, :, :, 0]).reshape(-1)[:n]
    invb = jnp.broadcast_to((1.0 / jnp.maximum(cnt, 1.0))[:, None], (n, 32))

    sc_passA = _make_sc_passA(n, e)

    wnode_all = jnp.concatenate([
        v1_w,
        jnp.concatenate([v2_w[:, :, 0:16], v4_w[:, :, 0:16]], axis=2),
        jnp.concatenate([v2_w[:, :, 16:32], v4_w[:, :, 16:32]], axis=2),
        v3_w[:, :, 0:16], v3_w[:, :, 16:32]], axis=2)
    bnode_all = jnp.concatenate([
        v1_b,
        v2_b[:, 0:16], v4_b[:, 0:16], v2_b[:, 16:32], v4_b[:, 16:32],
        v3_b[:, 0:16], v3_b[:, 16:32]], axis=1)
    bnode_all = jnp.concatenate(
        [bnode_all[:, None, :],
         jnp.zeros((d, 7, 128), jnp.float32)], axis=1)
    e0w_next = jnp.roll(e0_w, -1, axis=0)
    e0b_next = jnp.roll(e0_b, -1, axis=0)

    inv_e = 1.0 / e

    def layer(carry, xs):
        h, w_st, p_st = carry
        (wnode, bnode, vg, vb, eg, eb, ewn, ebn) = xs

        x1, t_st, r_st = _node1(h, wnode, bnode)
        agg2, stats, pre_st, _ = sc_passA(src2d, dst2d, w_st, p_st,
                                          t_st, r_st, zeros_n)
        agg2 = agg2.reshape(2, agg2.shape[1] * agg2.shape[2], LANES)
        h = _node2(x1, agg2[:, :n, :], invb, h, vg, vb)

        ssum = jnp.sum(stats[:, :, 0, :], axis=1).reshape(32)
        ssq = jnp.sum(stats[:, :, 1, :], axis=1).reshape(32)
        mu = ssum * inv_e
        var = ssq * inv_e - mu * mu
        inv_sig = lax.rsqrt(var + 1e-5)
        scale = inv_sig * eg
        shift = eb - mu * scale
        w_st, p_st = _passb(w_st, pre_st, scale, shift, ewn, ebn)
        return (h, w_st, p_st), None

    (h, _, _), _ = lax.scan(
        layer, (h, w_st, p_st),
        (wnode_all, bnode_all, v_bn_g, v_bn_b, e_bn_g, e_bn_b,
         e0w_next, e0b_next))
    return h


# speedup vs baseline: 1.0882x; 1.0882x over previous
import functools

import jax
import jax.numpy as jnp
import numpy as np
from jax import lax
from jax.experimental import pallas as pl
from jax.experimental.pallas import tpu as pltpu
from jax.experimental.pallas import tpu_sc as plsc

NSC = 2
NSUB = 16
LANES = 16
CHUNK = 512
SUB = 128

_PACK_CH = np.arange(256) % 16 + (np.arange(256) // 128) * 16
_PACK_GRP = (np.arange(256) % 128) // 16


def _pack_vec(v):
    return v[_PACK_CH]


def _pack_mat(m):
    return m[_PACK_CH[:, None], _PACK_CH[None, :]] * (
        _PACK_GRP[:, None] == _PACK_GRP[None, :]).astype(jnp.float32)


def _pad8(a):
    return jnp.concatenate(
        [a[None, :], jnp.zeros((7, a.shape[0]), jnp.float32)], axis=0)


def _prolh_body(x_ref, w_ref, b_ref, o_ref):
    y = jnp.dot(x_ref[...], w_ref[...], preferred_element_type=jnp.float32)
    y = y + b_ref[0:1, :]
    o_ref[...] = y * jax.nn.sigmoid(y)


def _prologue_nodes(x, w, b):
    n = x.shape[0]
    blk = 2000
    return pl.pallas_call(
        _prolh_body,
        grid=(n // blk,),
        in_specs=[
            pl.BlockSpec((blk, 128), lambda i: (i, 0)),
            pl.BlockSpec((128, 32), lambda i: (0, 0)),
            pl.BlockSpec((8, 32), lambda i: (0, 0)),
        ],
        out_specs=pl.BlockSpec((blk, 32), lambda i: (i, 0)),
        out_shape=jax.ShapeDtypeStruct((n, 32), jnp.float32),
    )(x, w, _pad8(b))


def _prole_body(ea_ref, s_ref, par_ref, wbig_ref, wo_ref, po_ref):
    wcat = jnp.dot(ea_ref[...], s_ref[...], preferred_element_type=jnp.float32)
    wcat = wcat + par_ref[0:1, :]
    wcat = wcat * jax.nn.sigmoid(wcat)
    wo_ref[0] = wcat[:, 0:128]
    wo_ref[1] = wcat[:, 128:256]
    pn = jnp.dot(wcat, wbig_ref[...], preferred_element_type=jnp.float32)
    pn = pn + par_ref[1:2, :]
    po_ref[0] = pn[:, 0:128]
    po_ref[1] = pn[:, 128:256]


def _prologue_edges(ea8, ew_lin, eb_lin, ew0, eb0):
    rows = ea8.shape[0]
    blk = 1000
    s = (ew_lin[0, _PACK_CH][None, :]
         * (jnp.arange(8)[:, None] == _PACK_GRP[None, :])).astype(jnp.float32)
    par = jnp.concatenate([
        _pack_vec(eb_lin)[None, :], _pack_vec(eb0)[None, :],
        jnp.zeros((6, 256), jnp.float32)], axis=0)
    return pl.pallas_call(
        _prole_body,
        grid=(rows // blk,),
        in_specs=[
            pl.BlockSpec((blk, 8), lambda i: (i, 0)),
            pl.BlockSpec((8, 256), lambda i: (0, 0)),
            pl.BlockSpec((8, 256), lambda i: (0, 0)),
            pl.BlockSpec((256, 256), lambda i: (0, 0)),
        ],
        out_specs=[pl.BlockSpec((2, blk, 128), lambda i: (0, i, 0))] * 2,
        out_shape=[jax.ShapeDtypeStruct((2, rows, 128), jnp.float32)] * 2,
    )(ea8, s, par, _pack_mat(ew0))


def _node1_body(h_ref, w_ref, b_ref, x1_ref, t_ref, r_ref):
    y = jnp.dot(h_ref[...], w_ref[...], preferred_element_type=jnp.float32)
    y = y + b_ref[0:1, :]
    x1_ref[...] = y[:, 0:32]
    t_ref[0] = y[:, 32:64]
    t_ref[1] = y[:, 64:96]
    r_ref[0] = y[:, 96:112]
    r_ref[1] = y[:, 112:128]


def _node1(h, wnode, bnode):
    n = h.shape[0]
    blk = 2000
    return pl.pallas_call(
        _node1_body,
        grid=(n // blk,),
        in_specs=[
            pl.BlockSpec((blk, 32), lambda i: (i, 0)),
            pl.BlockSpec((32, 128), lambda i: (0, 0)),
            pl.BlockSpec((8, 128), lambda i: (0, 0)),
        ],
        out_specs=[
            pl.BlockSpec((blk, 32), lambda i: (i, 0)),
            pl.BlockSpec((2, blk, 32), lambda i: (0, i, 0)),
            pl.BlockSpec((2, blk, 16), lambda i: (0, i, 0)),
        ],
        out_shape=[
            jax.ShapeDtypeStruct((n, 32), jnp.float32),
            jax.ShapeDtypeStruct((2, n, 32), jnp.float32),
            jax.ShapeDtypeStruct((2, n, 16), jnp.float32),
        ],
    )(h, wnode, bnode)


def _node2a_body(x1_ref, a_ref, invb_ref, u_ref, acc_ref):
    agg = jnp.concatenate([a_ref[0], a_ref[1]], axis=1)
    u = x1_ref[...] + agg * invb_ref[...]
    u_ref[...] = u
    acc_ref[0, 0:1, :] = jnp.sum(u, axis=0, keepdims=True)
    acc_ref[0, 1:2, :] = jnp.sum(u * u, axis=0, keepdims=True)


def _node2b_body(u_ref, h_ref, par_ref, out_ref):
    z = u_ref[...] * par_ref[0:1, :] + par_ref[1:2, :]
    out_ref[...] = h_ref[...] + z * jax.nn.sigmoid(z)


def _node2(x1, agg2, invb, h, g, b):
    n = x1.shape[0]
    blk = 2000
    nb = n // blk
    u, acc = pl.pallas_call(
        _node2a_body,
        grid=(nb,),
        in_specs=[
            pl.BlockSpec((blk, 32), lambda i: (i, 0)),
            pl.BlockSpec((2, blk, 16), lambda i: (0, i, 0)),
            pl.BlockSpec((blk, 32), lambda i: (i, 0)),
        ],
        out_specs=[
            pl.BlockSpec((blk, 32), lambda i: (i, 0)),
            pl.BlockSpec((1, 8, 32), lambda i: (i, 0, 0)),
        ],
        out_shape=[
            jax.ShapeDtypeStruct((n, 32), jnp.float32),
            jax.ShapeDtypeStruct((nb, 8, 32), jnp.float32),
        ],
    )(x1, agg2, invb)
    tot = jnp.sum(acc[:, 0:2, :], axis=0)
    mu = tot[0] / n
    var = tot[1] / n - mu * mu
    inv_sig = lax.rsqrt(var + 1e-5)
    scale = inv_sig * g
    shift = b - mu * scale
    par = jnp.concatenate([scale[None, :], shift[None, :],
                           jnp.zeros((6, 32), jnp.float32)], axis=0)
    return pl.pallas_call(
        _node2b_body,
        grid=(nb,),
        in_specs=[
            pl.BlockSpec((blk, 32), lambda i: (i, 0)),
            pl.BlockSpec((blk, 32), lambda i: (i, 0)),
            pl.BlockSpec((8, 32), lambda i: (0, 0)),
        ],
        out_specs=pl.BlockSpec((blk, 32), lambda i: (i, 0)),
        out_shape=jax.ShapeDtypeStruct((n, 32), jnp.float32),
    )(u, h, par)


def _passb_body(w_ref, p_ref, par_ref, wbig_ref, wo_ref, po_ref):
    wcat = jnp.concatenate([w_ref[0], w_ref[1]], axis=1)
    zcat = jnp.concatenate([p_ref[0], p_ref[1]], axis=1)
    zcat = zcat * par_ref[0:1, :] + par_ref[1:2, :]
    wn = wcat + zcat * jax.nn.sigmoid(zcat)
    wo_ref[0] = wn[:, 0:128]
    wo_ref[1] = wn[:, 128:256]
    pn = jnp.dot(wn, wbig_ref[...], preferred_element_type=jnp.float32)
    pn = pn + par_ref[2:3, :]
    po_ref[0] = pn[:, 0:128]
    po_ref[1] = pn[:, 128:256]


def _passb(w_st, pre_st, scale, shift, ew, eb):
    rows = w_st.shape[1]
    blk = 1000
    par = jnp.concatenate([
        _pack_vec(scale)[None, :], _pack_vec(shift)[None, :],
        _pack_vec(eb)[None, :], jnp.zeros((5, 256), jnp.float32)], axis=0)
    return pl.pallas_call(
        _passb_body,
        grid=(rows // blk,),
        in_specs=[
            pl.BlockSpec((2, blk, 128), lambda i: (0, i, 0)),
            pl.BlockSpec((2, blk, 128), lambda i: (0, i, 0)),
            pl.BlockSpec((8, 256), lambda i: (0, 0)),
            pl.BlockSpec((256, 256), lambda i: (0, 0)),
        ],
        out_specs=[pl.BlockSpec((2, blk, 128), lambda i: (0, i, 0))] * 2,
        out_shape=[jax.ShapeDtypeStruct((2, rows, 128), jnp.float32)] * 2,
    )(w_st, pre_st, par, _pack_mat(ew))


def _sc_mesh():
    return plsc.VectorSubcoreMesh(core_axis_name="c", subcore_axis_name="s",
                                  num_cores=NSC, num_subcores=NSUB)


_SC_PARAMS = pltpu.CompilerParams(use_tc_tiling_on_sc=False)


def _npad(n):
    return ((n // NSUB + 7) // 8 * 8) * NSUB


def _make_sc_passA(n, e):
    total_chunks = e // CHUNK
    n_loop = (total_chunks + NSUB - 1) // NSUB
    nidx = CHUNK // SUB
    rows = CHUNK // 8
    npad = _npad(n)
    nhalf = npad // 2
    dump = nhalf

    scratch = [
        pltpu.VMEM((nidx, SUB), jnp.int32),
        pltpu.VMEM((nidx, SUB), jnp.int32),
        pltpu.VMEM((rows, 128), jnp.float32),
        pltpu.VMEM((rows, 128), jnp.float32),
        pltpu.VMEM((CHUNK, 2 * LANES), jnp.float32),
        pltpu.VMEM((CHUNK, LANES), jnp.float32),
        pltpu.VMEM((CHUNK, LANES), jnp.float32),
        pltpu.VMEM((rows, 128), jnp.float32),
        pltpu.VMEM((2, LANES), jnp.float32),
        pltpu.VMEM_SHARED((nhalf + 8, LANES), jnp.float32),
        pltpu.SemaphoreType.DMA,
        pltpu.SemaphoreType.DMA,
    ]
    out_type = [
        jax.ShapeDtypeStruct((NSC, 2, nhalf, LANES), jnp.float32),
        jax.ShapeDtypeStruct((NSC, NSUB, 2, LANES), jnp.float32),
        jax.ShapeDtypeStruct((NSC, e // 8, 128), jnp.float32),
        jax.ShapeDtypeStruct((NSC, e, LANES), jnp.float32),
    ]

    @functools.partial(pl.kernel, out_type=out_type, mesh=_sc_mesh(),
                       scratch_types=scratch, compiler_params=_SC_PARAMS)
    def sc_passA(src2d_hbm, dst2d_hbm, w_hbm, p_hbm, t_hbm, r_hbm, zeros_hbm,
                 agg_out, stats_out, pre_out, c_out,
                 src_v, dst_v, w_v, p_v, t_v, r_v, c_v, pre_v, st_v,
                 agg_s, sem, sem2):
        c = lax.axis_index("c")
        s = lax.axis_index("s")

        slab = nhalf // NSUB
        soff = pl.multiple_of(s * slab, slab)

        st_v[0, :] = jnp.zeros((LANES,), jnp.float32)
        st_v[1, :] = jnp.zeros((LANES,), jnp.float32)

        @pl.loop(0, 2)
        def _(h2):
            pltpu.sync_copy(zeros_hbm.at[pl.ds(soff, slab)],
                            agg_s.at[pl.ds(soff, slab)])

            @pl.when(s == 0)
            def _():
                pltpu.sync_copy(zeros_hbm.at[pl.ds(0, 8)],
                                agg_s.at[pl.ds(nhalf, 8)])
            plsc.subcore_barrier()
            lo = h2 * nhalf

            @pl.loop(0, n_loop)
            def _(k):
                m = s + k * NSUB

                @pl.when(m < total_chunks)
                def _():
                    row0 = pl.multiple_of(m * nidx, nidx)
                    prow = pl.multiple_of(m * rows, rows)
                    base = pl.multiple_of(m * CHUNK, CHUNK)

                    outw = []

                    @pl.when(h2 == 0)
                    def _():
                        ins = [
                            pltpu.async_copy(
                                src2d_hbm.at[pl.ds(row0, nidx)], src_v,
                                sem2),
                            pltpu.async_copy(
                                dst2d_hbm.at[pl.ds(row0, nidx)], dst_v,
                                sem2),
                            pltpu.async_copy(
                                w_hbm.at[c, pl.ds(prow, rows)], w_v, sem2),
                            pltpu.async_copy(
                                p_hbm.at[c, pl.ds(prow, rows)], p_v, sem2),
                        ]
                        for cp in ins:
                            cp.wait()
                        cps = []
                        for j in range(nidx):
                            cps.append(pltpu.async_copy(
                                t_hbm.at[c].at[dst_v.at[j]],
                                t_v.at[pl.ds(j * SUB, SUB)], sem))
                            cps.append(pltpu.async_copy(
                                r_hbm.at[c].at[src_v.at[j]],
                                r_v.at[pl.ds(j * SUB, SUB)], sem))
                        for cp in cps:
                            cp.wait()

                        def row_body(r, car2):
                            es, eq = car2
                            for jj in range(8):
                                i = r * 8 + jj
                                w0 = w_v[r, pl.ds(jj * LANES, LANES)]
                                sg = 1.0 / (1.0 + jnp.exp(-w0))
                                g2 = t_v[i, pl.ds(0, LANES)]
                                c_v[i, :] = sg * g2
                                g4 = t_v[i, pl.ds(LANES, LANES)]
                                pre = (p_v[r, pl.ds(jj * LANES, LANES)]
                                       + r_v[i, :] + g4)
                                pre_v[r, pl.ds(jj * LANES, LANES)] = pre
                                es = es + pre
                                eq = eq + pre * pre
                            return es, eq

                        zero = jnp.zeros((LANES,), jnp.float32)
                        es, eq = lax.fori_loop(0, rows, row_body,
                                               (zero, zero))
                        st_v[0, :] += es
                        st_v[1, :] += eq
                        outw.append(pltpu.async_copy(
                            pre_v, pre_out.at[c, pl.ds(prow, rows)], sem2))
                        outw.append(pltpu.async_copy(
                            c_v, c_out.at[c, pl.ds(base, CHUNK)], sem2))

                    @pl.when(h2 == 1)
                    def _():
                        ins1 = [
                            pltpu.async_copy(
                                src2d_hbm.at[pl.ds(row0, nidx)], src_v,
                                sem2),
                            pltpu.async_copy(
                                c_out.at[c, pl.ds(base, CHUNK)], c_v,
                                sem2),
                        ]
                        for cp in ins1:
                            cp.wait()

                    for j in range(nidx):
                        for l0 in range(0, SUB, LANES):
                            iv = src_v[j, pl.ds(l0, LANES)]
                            ok = jnp.logical_and(iv >= lo, iv < lo + nhalf)
                            dst_v[j, pl.ds(l0, LANES)] = jnp.where(
                                ok, iv - lo, dump)
                    scs = []
                    for j in range(nidx):
                        scs.append(pltpu.async_copy(
                            c_v.at[pl.ds(j * SUB, SUB)],
                            agg_s.at[dst_v.at[j]], sem, add=True))
                    for cp in scs:
                        cp.wait()

                    @pl.when(h2 == 0)
                    def _():
                        for cp in outw:
                            cp.wait()

            plsc.subcore_barrier()
            pltpu.sync_copy(agg_s.at[pl.ds(soff, slab)],
                            agg_out.at[c, h2, pl.ds(soff, slab)])
            plsc.subcore_barrier()

        pltpu.sync_copy(st_v, stats_out.at[c, s])

    return sc_passA


def _make_sc_count(n, e):
    NOCT = 8
    npad = _npad(n)
    nq = ((npad // NOCT + 127) // 128) * 128
    total_chunks = e // CHUNK
    n_loop = (total_chunks + NSC * NSUB - 1) // (NSC * NSUB)
    nidx = CHUNK // SUB

    scratch = [
        pltpu.VMEM((nidx, SUB), jnp.int32),
        pltpu.VMEM((nidx, SUB), jnp.int32),
        pltpu.VMEM((SUB, LANES), jnp.float32),
        pltpu.VMEM_SHARED((nq + 8, LANES), jnp.float32),
        pltpu.SemaphoreType.DMA,
    ]
    out_type = [jax.ShapeDtypeStruct((NSC, NOCT, nq, LANES), jnp.float32)]

    @functools.partial(pl.kernel, out_type=out_type, mesh=_sc_mesh(),
                       scratch_types=scratch, compiler_params=_SC_PARAMS)
    def sc_count(src2d_hbm, ones_hbm, zeros_hbm, cnt_out,
                 src_v, loc_v, ones_v, agg_s, semc):
        c = lax.axis_index("c")
        s = lax.axis_index("s")
        wid = c * NSUB + s
        pltpu.sync_copy(ones_hbm, ones_v)

        @pl.loop(0, NOCT)
        def _(q):
            @pl.when(s == 0)
            def _():
                pltpu.sync_copy(zeros_hbm.at[pl.ds(0, nq + 8)],
                                agg_s.at[pl.ds(0, nq + 8)])
            plsc.subcore_barrier()
            qlo = q * nq

            @pl.loop(0, n_loop)
            def _(k):
                m = wid + k * (NSC * NSUB)

                @pl.when(m < total_chunks)
                def _():
                    moff = pl.multiple_of(m * nidx, nidx)
                    pltpu.sync_copy(src2d_hbm.at[pl.ds(moff, nidx)], src_v)
                    for j in range(nidx):
                        for l0 in range(0, SUB, LANES):
                            iv = src_v[j, pl.ds(l0, LANES)]
                            ok = jnp.logical_and(iv >= qlo, iv < qlo + nq)
                            loc = jnp.where(ok, iv - qlo, nq)
                            loc_v[j, pl.ds(l0, LANES)] = loc
                    scs = []
                    for j in range(nidx):
                        scs.append(pltpu.async_copy(
                            ones_v, agg_s.at[loc_v.at[j]], semc,
                            add=True))
                    for cp in scs:
                        cp.wait()

            plsc.subcore_barrier()

            @pl.when(s == 0)
            def _():
                pltpu.sync_copy(agg_s.at[pl.ds(0, nq)], cnt_out.at[c, q])
            plsc.subcore_barrier()

    return sc_count, nq


def kernel(x, edge_index, edge_attr, v_lin0_w, v_lin0_b, v1_w, v1_b, v2_w, v2_b, v3_w, v3_b, v4_w, v4_b, e_lin0_w, e_lin0_b, e0_w, e0_b, v_bn_g, v_bn_b, e_bn_g, e_bn_b):
    n = x.shape[0]
    e = edge_index.shape[1]
    d = v1_w.shape[0]

    src2d = edge_index[0].reshape(e // SUB, SUB)
    dst2d = edge_index[1].reshape(e // SUB, SUB)
    npad = _npad(n)
    zeros_n = jnp.zeros((npad, LANES), jnp.float32)
    ones_sub = jnp.ones((SUB, LANES), jnp.float32)

    h = _prologue_nodes(x, v_lin0_w, v_lin0_b)
    ea8 = edge_attr.reshape(e // 8, 8)
    w_st, p_st = _prologue_edges(ea8, e_lin0_w, e_lin0_b, e0_w[0], e0_b[0])

    sc_count, nq = _make_sc_count(n, e)
    cnt_o = sc_count(src2d, ones_sub, zeros_n)[0]
    cnt = (cnt_o[0, :, :, 0] + cnt_o[1, :, :, 0]).reshape(-1)[:n]
    invb = jnp.broadcast_to((1.0 / jnp.maximum(cnt, 1.0))[:, None], (n, 32))

    sc_passA = _make_sc_passA(n, e)

    wnode_all = jnp.concatenate([
        v1_w,
        jnp.concatenate([v2_w[:, :, 0:16], v4_w[:, :, 0:16]], axis=2),
        jnp.concatenate([v2_w[:, :, 16:32], v4_w[:, :, 16:32]], axis=2),
        v3_w[:, :, 0:16], v3_w[:, :, 16:32]], axis=2)
    bnode_all = jnp.concatenate([
        v1_b,
        v2_b[:, 0:16], v4_b[:, 0:16], v2_b[:, 16:32], v4_b[:, 16:32],
        v3_b[:, 0:16], v3_b[:, 16:32]], axis=1)
    bnode_all = jnp.concatenate(
        [bnode_all[:, None, :],
         jnp.zeros((d, 7, 128), jnp.float32)], axis=1)
    e0w_next = jnp.roll(e0_w, -1, axis=0)
    e0b_next = jnp.roll(e0_b, -1, axis=0)

    inv_e = 1.0 / e

    def layer(carry, xs):
        h, w_st, p_st = carry
        (wnode, bnode, vg, vb, eg, eb, ewn, ebn) = xs

        x1, t_st, r_st = _node1(h, wnode, bnode)
        agg2, stats, pre_st, _ = sc_passA(src2d, dst2d, w_st, p_st,
                                          t_st, r_st, zeros_n)
        agg2 = agg2.reshape(2, agg2.shape[1] * agg2.shape[2], LANES)
        h = _node2(x1, agg2[:, :n, :], invb, h, vg, vb)

        ssum = jnp.sum(stats[:, :, 0, :], axis=1).reshape(32)
        ssq = jnp.sum(stats[:, :, 1, :], axis=1).reshape(32)
        mu = ssum * inv_e
        var = ssq * inv_e - mu * mu
        inv_sig = lax.rsqrt(var + 1e-5)
        scale = inv_sig * eg
        shift = eb - mu * scale
        w_st, p_st = _passb(w_st, pre_st, scale, shift, ewn, ebn)
        return (h, w_st, p_st), None

    (h, _, _), _ = lax.scan(
        layer, (h, w_st, p_st),
        (wnode_all, bnode_all, v_bn_g, v_bn_b, e_bn_g, e_bn_b,
         e0w_next, e0b_next))
    return h
